# baseline (speedup 1.0000x reference)
import jax
import jax.numpy as jnp
from jax.experimental import pallas as pl
from jax.experimental.pallas import tpu as pltpu

N = 8192
D = 128
C = 2048
NB = N // C
INV = 1.0 / (N - 1)


def _body(nodes_ref, w2_ref, b2_ref, w3_ref, b3_ref, w5_ref, b5_ref,
          out_ref, s_ref, hs_ref, rhs_ref, c_ref, c25i_ref,
          nstash_ref, dn_ref, d_ref):
    p = pl.program_id(0)
    i = pl.program_id(1)

    @pl.when((p == 0) & (i == 0))
    def _init():
        s_ref[:] = jnp.zeros((D, D), jnp.float32)
        hs_ref[:] = jnp.zeros((1, D), jnp.float32)

    @pl.when(p == 0)
    def _accumulate():
        nc = nodes_ref[:]
        ncb = nc.astype(jnp.bfloat16)
        hc = jax.lax.dot_general(
            ncb, w3_ref[:].astype(jnp.bfloat16), (((1,), (1,)), ((), ())),
            preferred_element_type=jnp.float32) + b3_ref[:]
        hcb = hc.astype(jnp.bfloat16)
        s_ref[:] += jax.lax.dot_general(
            hcb, ncb, (((0,), (0,)), ((), ())),
            preferred_element_type=jnp.float32)
        hs_ref[:] += jnp.sum(hc, axis=0, keepdims=True)
        hsq = hcb * hcb
        d = jax.lax.dot_general(
            hsq, jnp.ones((D, D), jnp.bfloat16), (((1,), (0,)), ((), ())),
            preferred_element_type=jnp.float32)
        rows = pl.ds(i * C, C)
        nstash_ref[rows, :] = nc
        dn_ref[rows, :] = (d * nc).astype(jnp.bfloat16)
        d_ref[rows, :] = d.astype(jnp.bfloat16)

    @pl.when((p == 0) & (i == NB - 1))
    def _finalize():
        w25 = jax.lax.dot_general(
            w5_ref[:], w2_ref[:], (((1,), (0,)), ((), ())),
            preferred_element_type=jnp.float32)
        c25 = jax.lax.dot_general(
            b2_ref[:], w5_ref[:], (((1,), (1,)), ((), ())),
            preferred_element_type=jnp.float32)
        t = jax.lax.dot_general(
            s_ref[:], w25, (((1,), (1,)), ((), ())),
            preferred_element_type=jnp.float32) + jax.lax.dot_general(
            hs_ref[:], c25, (((0,), (0,)), ((), ())),
            preferred_element_type=jnp.float32)
        rhs_ref[0:D, :] = (jax.lax.dot_general(
            w3_ref[:], t, (((0,), (0,)), ((), ())),
            preferred_element_type=jnp.float32) * INV).astype(jnp.bfloat16)
        rhs_ref[D:2 * D, :] = (w25 * -INV).T.astype(jnp.bfloat16)
        c_ref[:] = jax.lax.dot_general(
            b3_ref[:], t, (((1,), (0,)), ((), ())),
            preferred_element_type=jnp.float32) * INV + b5_ref[:]
        c25i_ref[:] = c25 * INV

    @pl.when(p == 1)
    def _emit():
        rows = pl.ds(i * C, C)
        nc = nstash_ref[rows, :]
        lhs = jnp.concatenate([nc.astype(jnp.bfloat16), dn_ref[rows, :]],
                              axis=1)
        out_ref[:] = jax.lax.dot_general(
            lhs, rhs_ref[:], (((1,), (0,)), ((), ())),
            preferred_element_type=jnp.float32)
        out_ref[:] += nc + c_ref[:] - (
            d_ref[rows, :].astype(jnp.float32) * c25i_ref[:])


@jax.jit
def kernel(nodes_in, inputs, W2, b2, W3, b3, W5, b5):
    del inputs
    in_block = pl.BlockSpec(
        (C, D), lambda p, i: (jnp.where(p == 0, i, 0), 0))
    out_block = pl.BlockSpec(
        (C, D), lambda p, i: (jnp.where(p == 1, i, 0), 0))
    full_dd = pl.BlockSpec((D, D), lambda p, i: (0, 0))
    full_1d = pl.BlockSpec((1, D), lambda p, i: (0, 0))

    return pl.pallas_call(
        _body,
        grid=(2, NB),
        in_specs=[in_block, full_dd, full_1d, full_dd, full_1d,
                  full_dd, full_1d],
        out_specs=out_block,
        out_shape=jax.ShapeDtypeStruct((N, D), jnp.float32),
        scratch_shapes=[
            pltpu.VMEM((D, D), jnp.float32),
            pltpu.VMEM((1, D), jnp.float32),
            pltpu.VMEM((2 * D, D), jnp.bfloat16),
            pltpu.VMEM((1, D), jnp.float32),
            pltpu.VMEM((1, D), jnp.float32),
            pltpu.VMEM((N, D), jnp.float32),
            pltpu.VMEM((N, D), jnp.bfloat16),
            pltpu.VMEM((N, D), jnp.bfloat16),
        ],
    )(nodes_in, W2, b2.reshape(1, D), W3, b3.reshape(1, D),
      W5, b5.reshape(1, D))


# speedup vs baseline: 1.1797x; 1.1797x over previous
import jax
import jax.numpy as jnp
from jax.experimental import pallas as pl
from jax.experimental.pallas import tpu as pltpu

N = 8192
D = 128
C = 4096
NB = N // C
INV = 1.0 / (N - 1)


def _body(nodes_ref, w2_ref, b2_ref, w3_ref, b3_ref, w5_ref, b5_ref,
          out_ref, s_ref, hs_ref, rhs_ref, c_ref, c25i_ref,
          nstash_ref, dn_ref, d_ref):
    p = pl.program_id(0)
    i = pl.program_id(1)

    @pl.when((p == 0) & (i == 0))
    def _init():
        s_ref[:] = jnp.zeros((D, D), jnp.float32)
        hs_ref[:] = jnp.zeros((1, D), jnp.float32)

    @pl.when(p == 0)
    def _accumulate():
        nc = nodes_ref[:]
        ncb = nc.astype(jnp.bfloat16)
        hc = jax.lax.dot_general(
            ncb, w3_ref[:].astype(jnp.bfloat16), (((1,), (1,)), ((), ())),
            preferred_element_type=jnp.float32) + b3_ref[:]
        hcb = hc.astype(jnp.bfloat16)
        s_ref[:] += jax.lax.dot_general(
            hcb, ncb, (((0,), (0,)), ((), ())),
            preferred_element_type=jnp.float32)
        hs_ref[:] += jnp.sum(hc, axis=0, keepdims=True)
        hsq = hcb * hcb
        d = jax.lax.dot_general(
            hsq, jnp.ones((D, D), jnp.bfloat16), (((1,), (0,)), ((), ())),
            preferred_element_type=jnp.float32)
        rows = pl.ds(i * C, C)
        nstash_ref[rows, :] = nc
        dn_ref[rows, :] = (d * nc).astype(jnp.bfloat16)
        d_ref[rows, :] = d.astype(jnp.bfloat16)

    @pl.when((p == 0) & (i == NB - 1))
    def _finalize():
        w25 = jax.lax.dot_general(
            w5_ref[:], w2_ref[:], (((1,), (0,)), ((), ())),
            preferred_element_type=jnp.float32)
        c25 = jax.lax.dot_general(
            b2_ref[:], w5_ref[:], (((1,), (1,)), ((), ())),
            preferred_element_type=jnp.float32)
        t = jax.lax.dot_general(
            s_ref[:], w25, (((1,), (1,)), ((), ())),
            preferred_element_type=jnp.float32) + jax.lax.dot_general(
            hs_ref[:], c25, (((0,), (0,)), ((), ())),
            preferred_element_type=jnp.float32)
        rhs_ref[0:D, :] = (jax.lax.dot_general(
            w3_ref[:], t, (((0,), (0,)), ((), ())),
            preferred_element_type=jnp.float32) * INV).astype(jnp.bfloat16)
        rhs_ref[D:2 * D, :] = (w25 * -INV).T.astype(jnp.bfloat16)
        c_ref[:] = jax.lax.dot_general(
            b3_ref[:], t, (((1,), (0,)), ((), ())),
            preferred_element_type=jnp.float32) * INV + b5_ref[:]
        c25i_ref[:] = c25 * INV

    @pl.when(p == 1)
    def _emit():
        rows = pl.ds(i * C, C)
        nc = nstash_ref[rows, :]
        lhs = jnp.concatenate([nc.astype(jnp.bfloat16), dn_ref[rows, :]],
                              axis=1)
        out_ref[:] = jax.lax.dot_general(
            lhs, rhs_ref[:], (((1,), (0,)), ((), ())),
            preferred_element_type=jnp.float32)
        out_ref[:] += nc + c_ref[:] - (
            d_ref[rows, :].astype(jnp.float32) * c25i_ref[:])


@jax.jit
def kernel(nodes_in, inputs, W2, b2, W3, b3, W5, b5):
    del inputs
    in_block = pl.BlockSpec(
        (C, D), lambda p, i: (jnp.where(p == 0, i, 0), 0))
    out_block = pl.BlockSpec(
        (C, D), lambda p, i: (jnp.where(p == 1, i, 0), 0))
    full_dd = pl.BlockSpec((D, D), lambda p, i: (0, 0))
    full_1d = pl.BlockSpec((1, D), lambda p, i: (0, 0))

    return pl.pallas_call(
        _body,
        grid=(2, NB),
        in_specs=[in_block, full_dd, full_1d, full_dd, full_1d,
                  full_dd, full_1d],
        out_specs=out_block,
        out_shape=jax.ShapeDtypeStruct((N, D), jnp.float32),
        scratch_shapes=[
            pltpu.VMEM((D, D), jnp.float32),
            pltpu.VMEM((1, D), jnp.float32),
            pltpu.VMEM((2 * D, D), jnp.bfloat16),
            pltpu.VMEM((1, D), jnp.float32),
            pltpu.VMEM((1, D), jnp.float32),
            pltpu.VMEM((N, D), jnp.float32),
            pltpu.VMEM((N, D), jnp.bfloat16),
            pltpu.VMEM((N, D), jnp.bfloat16),
        ],
    )(nodes_in, W2, b2.reshape(1, D), W3, b3.reshape(1, D),
      W5, b5.reshape(1, D))


# speedup vs baseline: 1.1837x; 1.0034x over previous
import jax
import jax.numpy as jnp
from jax.experimental import pallas as pl
from jax.experimental.pallas import tpu as pltpu

N = 8192
D = 128
INV = 1.0 / (N - 1)


def _body(nodes_ref, w2_ref, b2_ref, w3_ref, b3_ref, w5_ref, b5_ref,
          out_ref):
    nc = nodes_ref[:]
    ncb = nc.astype(jnp.bfloat16)
    hc = jax.lax.dot_general(
        ncb, w3_ref[:].astype(jnp.bfloat16), (((1,), (1,)), ((), ())),
        preferred_element_type=jnp.float32) + b3_ref[:]
    hcb = hc.astype(jnp.bfloat16)
    s = jax.lax.dot_general(
        hcb, ncb, (((0,), (0,)), ((), ())),
        preferred_element_type=jnp.float32)
    hs = jnp.sum(hc, axis=0, keepdims=True)
    hsq = hcb * hcb
    d = jax.lax.dot_general(
        hsq, jnp.ones((D, D), jnp.bfloat16), (((1,), (0,)), ((), ())),
        preferred_element_type=jnp.float32)

    w25 = jax.lax.dot_general(
        w5_ref[:], w2_ref[:], (((1,), (0,)), ((), ())),
        preferred_element_type=jnp.float32)
    c25 = jax.lax.dot_general(
        b2_ref[:], w5_ref[:], (((1,), (1,)), ((), ())),
        preferred_element_type=jnp.float32)
    t = jax.lax.dot_general(
        s, w25, (((1,), (1,)), ((), ())),
        preferred_element_type=jnp.float32) + jax.lax.dot_general(
        hs, c25, (((0,), (0,)), ((), ())),
        preferred_element_type=jnp.float32)
    u = jax.lax.dot_general(
        w3_ref[:], t, (((0,), (0,)), ((), ())),
        preferred_element_type=jnp.float32) * INV
    c = jax.lax.dot_general(
        b3_ref[:], t, (((1,), (0,)), ((), ())),
        preferred_element_type=jnp.float32) * INV + b5_ref[:]

    lhs = jnp.concatenate([ncb, (d * nc).astype(jnp.bfloat16)], axis=1)
    rhs = jnp.concatenate(
        [u.astype(jnp.bfloat16), (w25 * -INV).T.astype(jnp.bfloat16)],
        axis=0)
    out_ref[:] = jax.lax.dot_general(
        lhs, rhs, (((1,), (0,)), ((), ())),
        preferred_element_type=jnp.float32)
    out_ref[:] += nc + c - d * (c25 * INV)


@jax.jit
def kernel(nodes_in, inputs, W2, b2, W3, b3, W5, b5):
    del inputs
    full_nd = pl.BlockSpec((N, D), lambda: (0, 0))
    full_dd = pl.BlockSpec((D, D), lambda: (0, 0))
    full_1d = pl.BlockSpec((1, D), lambda: (0, 0))

    return pl.pallas_call(
        _body,
        grid=(),
        in_specs=[full_nd, full_dd, full_1d, full_dd, full_1d,
                  full_dd, full_1d],
        out_specs=full_nd,
        out_shape=jax.ShapeDtypeStruct((N, D), jnp.float32),
    )(nodes_in, W2, b2.reshape(1, D), W3, b3.reshape(1, D),
      W5, b5.reshape(1, D))
